# Initial kernel scaffold; baseline (speedup 1.0000x reference)
import jax
import jax.numpy as jnp
from jax.experimental import pallas as pl


def kernel(edge_index, edge_type, basis1, comp1, root1, bias1, basis2, comp2, root2, bias2):
    raise NotImplementedError("write your pallas kernel here")


# speedup vs baseline: 37.9309x; 37.9309x over previous
import jax
import jax.numpy as jnp
from jax import lax
from jax.experimental import pallas as pl
from jax.experimental.pallas import tpu as pltpu
from jax.experimental.pallas import tpu_sc as plsc

NC = 2
NS = 16
NW = NC * NS
SZ = 128
GPC = 16
K = SZ * GPC
NCHUNK = 25
EPW = K * NCHUNK
EPAD = NW * EPW
CPAD = 1408
NPADT = 48


def _make_count_body(cntp):
    cpt = cntp // NS

    def body(skey_hbm, zc_hbm, ones_hbm, out_hbm, sk_v, ones_v, cnt_sh):
        cid = lax.axis_index("c")
        sid = lax.axis_index("s")
        wid = cid * NS + sid
        pltpu.sync_copy(zc_hbm, cnt_sh.at[pl.ds(sid * cpt, cpt)])
        pltpu.sync_copy(ones_hbm, ones_v)
        plsc.subcore_barrier()

        def chunk(j, carry):
            row0 = wid * (NCHUNK * GPC) + j * GPC
            pltpu.sync_copy(skey_hbm.at[pl.ds(row0, GPC)], sk_v)

            def group(m, c2):
                off = pl.multiple_of(m * SZ, SZ)
                pltpu.sync_copy(ones_v.at[pl.ds(off, SZ)],
                                cnt_sh.at[sk_v.at[m, 0]], add=True)
                return c2

            lax.fori_loop(0, GPC, group, 0)
            return carry

        lax.fori_loop(0, NCHUNK, chunk, 0)
        plsc.subcore_barrier()
        pltpu.sync_copy(cnt_sh.at[pl.ds(sid * cpt, cpt)],
                        out_hbm.at[cid, pl.ds(sid * cpt, cpt)])

    return body


def _make_edge_body(npad):
    rpt = npad // NS

    def body(table_hbm, gkey_hbm, skey_hbm, dst_hbm, inv_hbm, zr_hbm, out_hbm,
             gk_v, sk_v, d_v, rows_v, s_v, acc_sh, gsem, ssem):
        cid = lax.axis_index("c")
        sid = lax.axis_index("s")
        wid = cid * NS + sid
        pltpu.sync_copy(zr_hbm, acc_sh.at[pl.ds(sid * rpt, rpt)])
        plsc.subcore_barrier()

        def chunk(j, carry):
            row0 = wid * (NCHUNK * GPC) + j * GPC
            pltpu.sync_copy(gkey_hbm.at[pl.ds(row0, GPC)], gk_v)
            pltpu.sync_copy(skey_hbm.at[pl.ds(row0, GPC)], sk_v)
            pltpu.sync_copy(dst_hbm.at[pl.ds(row0, GPC)], d_v)

            def group(m, c2):
                off = pl.multiple_of(m * SZ, SZ)
                cpg = pltpu.async_copy(
                    table_hbm.at[gk_v.at[m, 0]],
                    rows_v.at[pl.ds(off, SZ)], gsem)
                cps = pltpu.async_copy(
                    inv_hbm.at[sk_v.at[m, 0]],
                    s_v.at[pl.ds(off, SZ)], ssem)
                cpg.wait()
                cps.wait()

                def scale(t, c3):
                    base = pl.multiple_of(off + t * 16, 16)
                    sv = s_v[pl.ds(base, 16)]
                    for i in range(16):
                        rows_v[base + i] = rows_v[base + i] * sv[i]
                    return c3

                lax.fori_loop(0, SZ // 16, scale, 0)
                pltpu.sync_copy(rows_v.at[pl.ds(off, SZ)],
                                acc_sh.at[d_v.at[m, 0]], add=True)
                return c2

            lax.fori_loop(0, GPC, group, 0)
            return carry

        lax.fori_loop(0, NCHUNK, chunk, 0)
        plsc.subcore_barrier()
        pltpu.sync_copy(acc_sh.at[pl.ds(sid * rpt, rpt)],
                        out_hbm.at[cid, pl.ds(sid * rpt, rpt)])

    return body


def kernel(edge_index, edge_type, basis1, comp1, root1, bias1,
           basis2, comp2, root2, bias2):
    N, H = root1.shape
    R, NB = comp1.shape
    C = root2.shape[1]
    E = edge_type.shape[0]
    f32 = jnp.float32
    mesh = plsc.VectorSubcoreMesh(core_axis_name="c", subcore_axis_name="s")

    cntp = R * N + CPAD
    npad = N + NPADT
    pade = EPAD - E

    src = edge_index[0]
    dst = edge_index[1]
    et = edge_type
    i32 = jnp.int32

    def pad3d(key, fill):
        keyp = jnp.concatenate(
            [key, jnp.full((pade,), fill, i32)])
        return keyp.reshape(EPAD // SZ, 1, SZ)

    gkey1 = pad3d(et * N + src, 0)
    gkey2 = pad3d(src * R + et, 0)
    skey = pad3d(et * N + dst, R * N)
    dst3d = pad3d(dst, npad - 1)

    cpt = cntp // NS
    rpt = npad // NS
    z_cnt = jnp.zeros((cpt,), f32)
    z_acc = jnp.zeros((rpt, H), f32)
    ones_k = jnp.ones((K,), f32)

    cnt_call = pl.kernel(
        _make_count_body(cntp),
        out_type=jax.ShapeDtypeStruct((NC, cntp), f32),
        mesh=mesh,
        compiler_params=pltpu.CompilerParams(use_tc_tiling_on_sc=False),
        scratch_types=[
            pltpu.VMEM((GPC, 1, SZ), jnp.int32),
            pltpu.VMEM((K,), f32),
            pltpu.VMEM_SHARED((cntp,), f32),
        ],
    )
    cnt_p = cnt_call(skey, z_cnt, ones_k)

    def _inv_body(c_ref, o_ref):
        o_ref[...] = (1.0 / jnp.maximum(c_ref[0] + c_ref[1], 1.0))[None]

    cblk = cntp // 49
    inv_cnt = pl.pallas_call(
        _inv_body,
        grid=(49,),
        in_specs=[pl.BlockSpec((2, cblk), lambda i: (0, i))],
        out_specs=pl.BlockSpec((1, cblk), lambda i: (0, i)),
        out_shape=jax.ShapeDtypeStruct((1, cntp), f32),
    )(cnt_p).reshape(cntp)

    def _w1_body(c_ref, b_ref, o_ref):
        o_ref[...] = jnp.dot(c_ref[...], b_ref[...],
                             preferred_element_type=f32)

    nh = N * H
    wblk = nh // 25
    w1 = pl.pallas_call(
        _w1_body,
        grid=(25,),
        in_specs=[pl.BlockSpec((R, NB), lambda i: (0, 0)),
                  pl.BlockSpec((NB, wblk), lambda i: (0, i))],
        out_specs=pl.BlockSpec((R, wblk), lambda i: (0, i)),
        out_shape=jax.ShapeDtypeStruct((R, nh), f32),
    )(comp1, basis1.reshape(NB, nh))
    table1 = w1.reshape(R * N, H)

    edge_call = pl.kernel(
        _make_edge_body(npad),
        out_type=jax.ShapeDtypeStruct((NC, npad, H), f32),
        mesh=mesh,
        compiler_params=pltpu.CompilerParams(use_tc_tiling_on_sc=False),
        scratch_types=[
            pltpu.VMEM((GPC, 1, SZ), jnp.int32),
            pltpu.VMEM((GPC, 1, SZ), jnp.int32),
            pltpu.VMEM((GPC, 1, SZ), jnp.int32),
            pltpu.VMEM((K, H), f32),
            pltpu.VMEM((K,), f32),
            pltpu.VMEM_SHARED((npad, H), f32),
            pltpu.SemaphoreType.DMA,
            pltpu.SemaphoreType.DMA,
        ],
    )
    acc1_p = edge_call(table1, gkey1, skey, dst3d, inv_cnt, z_acc)

    w2 = (comp2 @ basis2.reshape(NB, H * C)).reshape(R, H, C)
    w2s = w2.transpose(1, 0, 2).reshape(H, R * C)
    rb1 = root1 + bias1[None, :]

    def _x_xw_body(p_ref, rb_ref, w2_ref, x_ref, xw_ref):
        xb = jnp.maximum(p_ref[0] + p_ref[1] + rb_ref[...], 0.0)
        x_ref[...] = xb
        xw_ref[...] = jnp.dot(xb, w2_ref[...], preferred_element_type=f32)

    nblk = N // 25
    x, xw = pl.pallas_call(
        _x_xw_body,
        grid=(25,),
        in_specs=[pl.BlockSpec((2, nblk, H), lambda i: (0, i, 0)),
                  pl.BlockSpec((nblk, H), lambda i: (i, 0)),
                  pl.BlockSpec((H, R * C), lambda i: (0, 0))],
        out_specs=[pl.BlockSpec((nblk, H), lambda i: (i, 0)),
                   pl.BlockSpec((nblk, R * C), lambda i: (i, 0))],
        out_shape=[jax.ShapeDtypeStruct((N, H), f32),
                   jax.ShapeDtypeStruct((N, R * C), f32)],
    )(acc1_p, rb1, w2s)
    table2 = xw.reshape(N * R, C)

    acc2_p = edge_call(table2, gkey2, skey, dst3d, inv_cnt, z_acc)

    def _out_body(p_ref, x_ref, r2_ref, b2_ref, o_ref):
        o = (p_ref[0] + p_ref[1] + b2_ref[...]
             + jnp.dot(x_ref[...], r2_ref[...], preferred_element_type=f32))
        m = jnp.max(o, axis=-1, keepdims=True)
        s = o - m
        o_ref[...] = s - jnp.log(jnp.sum(jnp.exp(s), axis=-1, keepdims=True))

    out = pl.pallas_call(
        _out_body,
        grid=(25,),
        in_specs=[pl.BlockSpec((2, nblk, C), lambda i: (0, i, 0)),
                  pl.BlockSpec((nblk, H), lambda i: (i, 0)),
                  pl.BlockSpec((H, C), lambda i: (0, 0)),
                  pl.BlockSpec((1, C), lambda i: (0, 0))],
        out_specs=pl.BlockSpec((nblk, C), lambda i: (i, 0)),
        out_shape=jax.ShapeDtypeStruct((N, C), f32),
    )(acc2_p, x, root2, bias2[None, :])
    return out
